# Initial kernel scaffold; baseline (speedup 1.0000x reference)
import jax
import jax.numpy as jnp
from jax.experimental import pallas as pl


def kernel(x, edge_index, eps, W1, b1, W2, b2, gamma, beta):
    raise NotImplementedError("write your pallas kernel here")


# speedup vs baseline: 6.4264x; 6.4264x over previous
import functools

import jax
import jax.numpy as jnp
from jax import lax
from jax.experimental import pallas as pl
from jax.experimental.pallas import tpu as pltpu
from jax.experimental.pallas import tpu_sc as plsc

N = 10000
E = 320000
D = 128
NC = 2
NS = 16
C = 80
NCHUNK = E // (NC * NS * C)
SUB_STRIDE = 624
SUB_ROWS = 640


def _segsum_body(h_hbm, srcr, dstr, zeros_hbm, out_hbm,
                 src_v, dst_v, rows_v, acc, sem):
    c = lax.axis_index("c")
    s = lax.axis_index("s")

    row0 = s * SUB_STRIDE
    pltpu.sync_copy(zeros_hbm.at[pl.ds(row0, SUB_ROWS)],
                    acc.at[pl.ds(row0, SUB_ROWS)])

    pltpu.sync_copy(srcr.at[c, s], src_v)
    pltpu.sync_copy(dstr.at[c, s], dst_v)
    plsc.subcore_barrier()

    def body(j, carry):
        pltpu.async_copy(h_hbm.at[src_v.at[j]], rows_v, sem).wait()
        pltpu.sync_copy(rows_v, acc.at[dst_v.at[j]], add=True)
        return carry

    lax.fori_loop(0, NCHUNK, body, 0)
    plsc.subcore_barrier()

    pltpu.sync_copy(acc.at[pl.ds(row0, SUB_ROWS)],
                    out_hbm.at[c, pl.ds(row0, SUB_ROWS)])


_segsum_sc = pl.kernel(
    _segsum_body,
    out_type=jax.ShapeDtypeStruct((NC, N, D), jnp.float32),
    mesh=plsc.VectorSubcoreMesh(core_axis_name="c", subcore_axis_name="s"),
    scratch_types=[
        pltpu.VMEM((NCHUNK, C), jnp.int32),
        pltpu.VMEM((NCHUNK, C), jnp.int32),
        pltpu.VMEM((C, D), jnp.float32),
        pltpu.VMEM_SHARED((N, D), jnp.float32),
        pltpu.SemaphoreType.DMA,
    ],
)


def _layer_body(eps_ref, p_ref, h_ref, w1_ref, b1_ref, w2_ref, b2_ref,
                g_ref, bt_ref, out_ref):
    pooled = p_ref[0] + p_ref[1] + (1.0 + eps_ref[0]) * h_ref[...]
    t = jnp.dot(pooled, w1_ref[...], preferred_element_type=jnp.float32)
    t = jnp.maximum(t + b1_ref[...], 0.0)
    t = jnp.dot(t, w2_ref[...], preferred_element_type=jnp.float32)
    t = t + b2_ref[...]
    mu = jnp.mean(t, axis=0, keepdims=True)
    ctr = t - mu
    var = jnp.mean(ctr * ctr, axis=0, keepdims=True)
    t = g_ref[...] * ctr * lax.rsqrt(var + 1e-5) + bt_ref[...]
    out_ref[...] = jnp.maximum(t, 0.0)


@functools.partial(jax.jit, static_argnames=())
def _layer_tc(eps_l, p, h, w1, b1, w2, b2, g, bt):
    return pl.pallas_call(
        _layer_body,
        out_shape=jax.ShapeDtypeStruct((N, D), jnp.float32),
        in_specs=[
            pl.BlockSpec(memory_space=pltpu.SMEM),
            pl.BlockSpec(memory_space=pltpu.VMEM),
            pl.BlockSpec(memory_space=pltpu.VMEM),
            pl.BlockSpec(memory_space=pltpu.VMEM),
            pl.BlockSpec(memory_space=pltpu.VMEM),
            pl.BlockSpec(memory_space=pltpu.VMEM),
            pl.BlockSpec(memory_space=pltpu.VMEM),
            pl.BlockSpec(memory_space=pltpu.VMEM),
            pl.BlockSpec(memory_space=pltpu.VMEM),
        ],
        out_specs=pl.BlockSpec(memory_space=pltpu.VMEM),
    )(eps_l, p, h, w1, b1, w2, b2, g, bt)


def kernel(x, edge_index, eps, W1, b1, W2, b2, gamma, beta):
    L = W1.shape[0]
    srcr = edge_index[0].reshape(NC, NS, NCHUNK, C)
    dstr = edge_index[1].reshape(NC, NS, NCHUNK, C)
    zeros = jnp.zeros((N, D), jnp.float32)

    h = x
    outs = [h]
    for l in range(L):
        p = _segsum_sc(h, srcr, dstr, zeros)
        h = _layer_tc(eps[l].reshape(1), p, h,
                      W1[l], b1[l].reshape(1, D), W2[l], b2[l].reshape(1, D),
                      gamma[l].reshape(1, D), beta[l].reshape(1, D))
        outs.append(h)
    return jnp.stack(outs, axis=0)
